# baseline (speedup 1.0000x reference)
import jax
import jax.numpy as jnp
from jax.experimental import pallas as pl


BATCH = 4
SEQ_LEN = 2048
D_MODEL = 1024
SEQ_BLK = 256


def _add_kernel(x_ref, emb_ref, out_ref):
    out_ref[...] = x_ref[...] + emb_ref[...][None, :, :]


def kernel(x, emb):
    grid = (SEQ_LEN // SEQ_BLK,)
    return pl.pallas_call(
        _add_kernel,
        grid=grid,
        in_specs=[
            pl.BlockSpec((BATCH, SEQ_BLK, D_MODEL), lambda s: (0, s, 0)),
            pl.BlockSpec((SEQ_BLK, D_MODEL), lambda s: (s, 0)),
        ],
        out_specs=pl.BlockSpec((BATCH, SEQ_BLK, D_MODEL), lambda s: (0, s, 0)),
        out_shape=jax.ShapeDtypeStruct((BATCH, SEQ_LEN, D_MODEL), x.dtype),
    )(x, emb)
